# baseline (speedup 1.0000x reference)
import functools

import jax
import jax.numpy as jnp
from jax.experimental import pallas as pl
from jax.experimental.pallas import tpu as pltpu

_FAST = jax.lax.Precision.DEFAULT


def _sweep1_body(x_ref, adj_ref, w1_ref, b1_ref, w2_ref, b2_ref,
                 emb_ref, s2_ref, buf_ref, s1_ref, h_ref, *,
                 block, cw, n, nct):
    r = pl.program_id(0)
    c = pl.program_id(1)
    cs = (block * r) // cw

    @pl.when(jnp.logical_and(r == 0, c == 0))
    def _():
        s1_ref[...] = jnp.dot(x_ref[...], w1_ref[...],
                              preferred_element_type=jnp.float32)
        s2_ref[...] = jnp.zeros_like(s2_ref)
        emb_ref[...] = jnp.broadcast_to(b2_ref[...], emb_ref.shape)

    tail = n - (nct - 1) * cw
    off = pl.multiple_of(c * cw, cw)

    if tail < cw:
        last_w = tail
    else:
        last_w = cw

    def h_piece(width):
        def _():
            piece = jnp.dot(
                adj_ref[:, pl.ds(off, cw)] if width == cw
                else adj_ref[:, (nct - 1) * cw:n],
                s1_ref[pl.ds(off, cw), :] if width == cw
                else s1_ref[(nct - 1) * cw:n, :],
                precision=_FAST, preferred_element_type=jnp.float32)
            h_ref[...] = jnp.where(c == 0, piece, h_ref[...] + piece)
        return _

    if tail < cw:
        pl.when(c < nct - 1)(h_piece(cw))
        pl.when(c == nct - 1)(h_piece(last_w))
    else:
        h_piece(cw)()

    @pl.when(jnp.logical_and(c <= cs, r > 0))
    def _():
        piece = jnp.dot(
            adj_ref[:, pl.ds(off, cw)],
            s2_ref[pl.ds(off, cw), :],
            precision=_FAST, preferred_element_type=jnp.float32)
        emb_ref[pl.ds(r * block, block), :] += piece

    @pl.when(c == nct - 1)
    def _():
        hrow = jnp.maximum(h_ref[...] + b1_ref[...], 0.0)
        s2_ref[pl.ds(r * block, block), :] = jnp.dot(
            hrow, w2_ref[...], preferred_element_type=jnp.float32)

    @pl.when(jnp.logical_and(c >= cs, c < (nct - 1 if tail < cw else nct)))
    def _():
        buf_ref[0, 0, :, :] = adj_ref[:, pl.ds(off, cw)].astype(jnp.bfloat16)

    if tail < cw:
        @pl.when(c == nct - 1)
        def _():
            part = adj_ref[:, (nct - 1) * cw:n].astype(jnp.bfloat16)
            buf_ref[0, 0, :, :] = jnp.concatenate(
                [part, jnp.zeros((block, cw - tail), jnp.bfloat16)], axis=1)


def _sweep2_body(buf_ref, s2b_ref, embp_ref, out_ref, *, block, cw):
    r = pl.program_id(0)
    c = pl.program_id(1)
    cs = (block * r) // cw

    @pl.when(jnp.logical_and(r == 0, c == 0))
    def _():
        out_ref[...] = embp_ref[...]

    s2_slice = s2b_ref[pl.ds(pl.multiple_of(c * cw, cw), cw), :]

    @pl.when(jnp.logical_and(c == cs, block * r > cw * cs))
    def _():
        row_ids = c * cw + jax.lax.broadcasted_iota(
            jnp.int32, s2_slice.shape, 0)
        s2m = jnp.where(row_ids >= block * r, s2_slice,
                        jnp.zeros_like(s2_slice))
        out_ref[pl.ds(r * block, block), :] += jnp.dot(
            buf_ref[0, 0, :, :], s2m, preferred_element_type=jnp.float32)

    @pl.when(jnp.logical_or(c > cs,
                            jnp.logical_and(c == cs, block * r <= cw * cs)))
    def _():
        out_ref[pl.ds(r * block, block), :] += jnp.dot(
            buf_ref[0, 0, :, :], s2_slice,
            preferred_element_type=jnp.float32)


def kernel(x, adj, W1, b1, W2, b2):
    n, nfeat = x.shape
    hid1 = W1.shape[1]
    nout = W2.shape[1]

    block = next(b for b in (200, 100, 40, 8, 1) if n % b == 0)
    nblocks = n // block
    cw = 2048 if n >= 2048 else 64 if n >= 64 else 8
    nct = -(-n // cw)

    b1r = b1.reshape(1, hid1)
    b2r = b2.reshape(1, nout)

    def buf_index(r, c):
        return (r, jnp.maximum(c, (block * r) // cw), 0, 0)

    emb_part, s2, buf = pl.pallas_call(
        functools.partial(_sweep1_body, block=block, cw=cw, n=n, nct=nct),
        grid=(nblocks, nct),
        in_specs=[
            pl.BlockSpec((n, nfeat), lambda r, c: (0, 0)),
            pl.BlockSpec((block, n), lambda r, c: (r, 0)),
            pl.BlockSpec((nfeat, hid1), lambda r, c: (0, 0)),
            pl.BlockSpec((1, hid1), lambda r, c: (0, 0)),
            pl.BlockSpec((hid1, nout), lambda r, c: (0, 0)),
            pl.BlockSpec((1, nout), lambda r, c: (0, 0)),
        ],
        out_specs=[
            pl.BlockSpec((n, nout), lambda r, c: (0, 0)),
            pl.BlockSpec((n, nout), lambda r, c: (0, 0)),
            pl.BlockSpec((1, 1, block, cw), buf_index),
        ],
        out_shape=[
            jax.ShapeDtypeStruct((n, nout), jnp.float32),
            jax.ShapeDtypeStruct((n, nout), jnp.float32),
            jax.ShapeDtypeStruct((nblocks, nct, block, cw), jnp.bfloat16),
        ],
        scratch_shapes=[
            pltpu.VMEM((n, hid1), jnp.float32),
            pltpu.VMEM((block, hid1), jnp.float32),
        ],
        compiler_params=pltpu.CompilerParams(
            dimension_semantics=("arbitrary", "arbitrary"),
        ),
    )(x, adj, W1, b1r, W2, b2r)

    s2b = jnp.pad(s2, ((0, nct * cw - n), (0, 0))).astype(jnp.bfloat16)

    out = pl.pallas_call(
        functools.partial(_sweep2_body, block=block, cw=cw),
        grid=(nblocks, nct),
        in_specs=[
            pl.BlockSpec((1, 1, block, cw), buf_index),
            pl.BlockSpec((nct * cw, nout), lambda r, c: (0, 0)),
            pl.BlockSpec((n, nout), lambda r, c: (0, 0)),
        ],
        out_specs=pl.BlockSpec((n, nout), lambda r, c: (0, 0)),
        out_shape=jax.ShapeDtypeStruct((n, nout), jnp.float32),
        compiler_params=pltpu.CompilerParams(
            dimension_semantics=("arbitrary", "arbitrary"),
        ),
    )(buf, s2b, emb_part)
    return out


# speedup vs baseline: 2.1882x; 2.1882x over previous
import functools

import jax
import jax.numpy as jnp
from jax.experimental import pallas as pl
from jax.experimental.pallas import tpu as pltpu

_FAST = jax.lax.Precision.DEFAULT


def _gcn_body(x_ref, adj_ref, w1_ref, b1_ref, w2_ref, b2_ref,
              out_ref, s1_ref, s2_ref, *, block):
    p = pl.program_id(0)
    i = pl.program_id(1)

    @pl.when(jnp.logical_and(p == 0, i == 0))
    def _():
        s1_ref[...] = jnp.dot(x_ref[...], w1_ref[...], precision=_FAST,
                              preferred_element_type=jnp.float32)

    @pl.when(p == 0)
    def _():
        h = jnp.dot(adj_ref[...], s1_ref[...], precision=_FAST,
                    preferred_element_type=jnp.float32)
        h = jnp.maximum(h + b1_ref[...], 0.0)
        s2_ref[pl.ds(i * block, block), :] = jnp.dot(
            h, w2_ref[...], precision=_FAST,
            preferred_element_type=jnp.float32)

    @pl.when(p == 1)
    def _():
        out_ref[...] = (
            jnp.dot(adj_ref[...], s2_ref[...], precision=_FAST,
                    preferred_element_type=jnp.float32)
            + b2_ref[...])


def kernel(x, adj, W1, b1, W2, b2):
    n, nfeat = x.shape
    hid1 = W1.shape[1]
    nout = W2.shape[1]

    block = next(b for b in (400, 200, 100, 50, 25, 20, 10, 8, 5, 4, 2, 1)
                 if n % b == 0)
    grid = (2, n // block)

    b1r = b1.reshape(1, hid1)
    b2r = b2.reshape(1, nout)

    out = pl.pallas_call(
        functools.partial(_gcn_body, block=block),
        grid=grid,
        in_specs=[
            pl.BlockSpec((n, nfeat), lambda p, i: (0, 0)),
            pl.BlockSpec((block, n), lambda p, i: (i, 0)),
            pl.BlockSpec((nfeat, hid1), lambda p, i: (0, 0)),
            pl.BlockSpec((1, hid1), lambda p, i: (0, 0)),
            pl.BlockSpec((hid1, nout), lambda p, i: (0, 0)),
            pl.BlockSpec((1, nout), lambda p, i: (0, 0)),
        ],
        out_specs=pl.BlockSpec((block, nout), lambda p, i: (i, 0)),
        out_shape=jax.ShapeDtypeStruct((n, nout), jnp.float32),
        scratch_shapes=[
            pltpu.VMEM((n, hid1), jnp.float32),
            pltpu.VMEM((n, nout), jnp.float32),
        ],
        compiler_params=pltpu.CompilerParams(
            dimension_semantics=("arbitrary", "arbitrary"),
        ),
    )(x, adj, W1, b1r, W2, b2r)
    return out
